# baseline (speedup 1.0000x reference)
import functools

import jax
import jax.numpy as jnp
from jax import lax
from jax.experimental import pallas as pl
from jax.experimental.pallas import tpu as pltpu
from jax.experimental.pallas import tpu_sc as plsc

N_NODES = 10000
N_PAD = 10240
N_EDGES = 320000
E_PAD = 327680
D_FEAT = 128
D_HID = 16
N_CLASSES = 7

NCORE = 2
NSUB = 16
NW = NCORE * NSUB
EPW = E_PAD // NW
NCHUNK = 80
CW = 128
NB = 8
NGROUP = NCHUNK // NB
RPT = N_PAD // NSUB

@functools.cache
def _get_deg_kernel():
    mesh = plsc.VectorSubcoreMesh(
        core_axis_name="c", subcore_axis_name="s",
        num_cores=NCORE, num_subcores=NSUB)
    return functools.partial(
        pl.kernel,
        out_type=jax.ShapeDtypeStruct((NCORE, N_PAD), jnp.float32),
        mesh=mesh,
        scratch_types=[
            pltpu.VMEM((EPW,), jnp.int32),
            pltpu.VMEM((N_PAD,), jnp.float32),
            pltpu.VMEM((RPT,), jnp.float32),
            pltpu.VMEM((RPT,), jnp.float32),
            pltpu.VMEM_SHARED((NSUB, N_PAD), jnp.float32),
        ],
        compiler_params=pltpu.CompilerParams(needs_layout_passes=False),
    )(_deg_body)


def _deg_body(dst_hbm, out_hbm, dstv, degl, tmp, accv, shared):
    c = lax.axis_index("c")
    s = lax.axis_index("s")
    wid = c * NSUB + s
    pltpu.sync_copy(dst_hbm.at[pl.ds(wid * EPW, EPW)], dstv)
    zeros16 = jnp.zeros((16,), jnp.float32)
    ones16 = jnp.ones((16,), jnp.float32)

    def zero_body(j, carry):
        degl[pl.ds(j * 16, 16)] = zeros16
        return carry
    lax.fori_loop(0, N_PAD // 16, zero_body, 0)

    def count_body(j, carry):
        idx = dstv[pl.ds(j * 16, 16)]
        plsc.addupdate_scatter(degl, [idx], ones16)
        return carry
    lax.fori_loop(0, EPW // 16, count_body, 0)

    pltpu.sync_copy(degl, shared.at[s])
    plsc.subcore_barrier()

    base = s * RPT
    pltpu.sync_copy(shared.at[0, pl.ds(base, RPT)], accv)

    def red_body(t, carry):
        pltpu.sync_copy(shared.at[t, pl.ds(base, RPT)], tmp)

        def add_body(j, carry2):
            accv[pl.ds(j * 16, 16)] = accv[pl.ds(j * 16, 16)] + tmp[pl.ds(j * 16, 16)]
            return carry2
        lax.fori_loop(0, RPT // 16, add_body, 0)
        return carry
    lax.fori_loop(1, NSUB, red_body, 0)

    pltpu.sync_copy(accv, out_hbm.at[c, pl.ds(base, RPT)])


@functools.cache
def _get_edge_kernel():
    mesh = plsc.VectorSubcoreMesh(
        core_axis_name="c", subcore_axis_name="s",
        num_cores=NCORE, num_subcores=NSUB)
    return functools.partial(
        pl.kernel,
        out_type=jax.ShapeDtypeStruct((NCORE, N_PAD, D_HID), jnp.float32),
        mesh=mesh,
        scratch_types=[
            pltpu.VMEM((NCHUNK, CW), jnp.int32),
            pltpu.VMEM((NCHUNK, CW), jnp.int32),
            pltpu.VMEM((2, NB, CW, D_HID), jnp.float32),
            pltpu.VMEM_SHARED((N_PAD, D_HID), jnp.float32),
            pltpu.SemaphoreType.DMA,
            pltpu.SemaphoreType.DMA,
        ],
        compiler_params=pltpu.CompilerParams(
            needs_layout_passes=False, use_tc_tiling_on_sc=False),
    )(_edge_body)


def _edge_body(g_hbm, src_hbm, dst_hbm, zrows_hbm, out_hbm,
               sidx, didx, rows, shared, sem_a, sem_b):
    c = lax.axis_index("c")
    s = lax.axis_index("s")
    wid = c * NSUB + s
    rbase = s * RPT

    pltpu.sync_copy(zrows_hbm, shared.at[pl.ds(rbase, RPT)])
    pltpu.sync_copy(src_hbm.at[wid], sidx)
    pltpu.sync_copy(dst_hbm.at[wid], didx)
    plsc.subcore_barrier()

    def _fire(g, half, sem):
        for b in range(NB):
            pltpu.async_copy(g_hbm.at[sidx.at[g * NB + b]],
                             rows.at[half, b], sem)

    def _drain_scatter(g, half, sem):
        for b in range(NB):
            pltpu.make_async_copy(g_hbm.at[pl.ds(0, CW)],
                                  rows.at[half, b], sem).wait()
            pltpu.sync_copy(rows.at[half, b],
                            shared.at[didx.at[g * NB + b]], add=True)

    _fire(0, 0, sem_a)

    def body(kk, carry):
        g0 = 2 * kk
        _fire(g0 + 1, 1, sem_b)
        _drain_scatter(g0, 0, sem_a)

        @pl.when(g0 + 2 < NGROUP)
        def _():
            _fire(g0 + 2, 0, sem_a)
        _drain_scatter(g0 + 1, 1, sem_b)
        return carry
    lax.fori_loop(0, NGROUP // 2, body, 0)

    plsc.subcore_barrier()
    pltpu.sync_copy(shared.at[pl.ds(rbase, RPT)],
                    out_hbm.at[c, pl.ds(rbase, RPT)])


def _tc_a_body(x_ref, w1_ref, degp_ref, g_ref, dis_ref):
    deg = degp_ref[:, 0:1] + degp_ref[:, 1:2] + 1.0
    dis = lax.rsqrt(deg)
    h = jnp.dot(x_ref[...], w1_ref[...], preferred_element_type=jnp.float32)
    g_ref[...] = h * dis
    dis_ref[...] = dis


_tc_a = pl.pallas_call(
    _tc_a_body,
    out_shape=(jax.ShapeDtypeStruct((N_PAD, D_HID), jnp.float32),
               jax.ShapeDtypeStruct((N_PAD, 1), jnp.float32)),
)


def _tc_b_body(accp_ref, g1_ref, dis_ref, b1_ref, w2_ref, g2_ref):
    acc = accp_ref[0] + accp_ref[1] + g1_ref[...]
    dis = dis_ref[...]
    t = jnp.maximum(acc * dis + b1_ref[...], 0.0)
    g2_ref[...] = jnp.dot(t, w2_ref[...],
                          preferred_element_type=jnp.float32) * dis


_tc_b = pl.pallas_call(
    _tc_b_body,
    out_shape=jax.ShapeDtypeStruct((N_PAD, D_HID), jnp.float32),
)


def _tc_c_body(accp_ref, g2_ref, dis_ref, b2_ref, out_ref):
    acc = accp_ref[0] + accp_ref[1] + g2_ref[...]
    z = acc * dis_ref[...] + b2_ref[...]
    col = lax.broadcasted_iota(jnp.int32, z.shape, 1)
    zm = jnp.where(col < N_CLASSES, z, -jnp.inf)
    m = jnp.max(zm, axis=1, keepdims=True)
    se = jnp.sum(jnp.exp(zm - m), axis=1, keepdims=True)
    out_ref[...] = z - m - jnp.log(se)


_tc_c = pl.pallas_call(
    _tc_c_body,
    out_shape=jax.ShapeDtypeStruct((N_PAD, D_HID), jnp.float32),
)


def kernel(x, edge_index, W1, b1, W2, b2):
    src = edge_index[0].astype(jnp.int32)
    dst = edge_index[1].astype(jnp.int32)
    padv = jnp.full((E_PAD - N_EDGES,), N_NODES, jnp.int32)
    src_flat = jnp.concatenate([src, padv])
    dst_flat = jnp.concatenate([dst, padv])
    src3 = src_flat.reshape(NW, NCHUNK, CW)
    dst3 = dst_flat.reshape(NW, NCHUNK, CW)

    xp = jnp.pad(x, ((0, N_PAD - N_NODES), (0, 0)))
    w2p = jnp.pad(W2, ((0, 0), (0, D_HID - N_CLASSES)))
    b1r = b1.reshape(1, D_HID)
    b2r = jnp.pad(b2, (0, D_HID - N_CLASSES)).reshape(1, D_HID)
    zrows = jnp.zeros((RPT, D_HID), jnp.float32)

    deg_kernel = _get_deg_kernel()
    edge_kernel = _get_edge_kernel()
    degp = deg_kernel(dst_flat)
    g1, dis = _tc_a(xp, W1, degp.T)
    accp1 = edge_kernel(g1, src3, dst3, zrows)
    g2 = _tc_b(accp1, g1, dis, b1r, w2p)
    accp2 = edge_kernel(g2, src3, dst3, zrows)
    z = _tc_c(accp2, g2, dis, b2r)
    return z[:N_NODES, :N_CLASSES]


# speedup vs baseline: 1.0073x; 1.0073x over previous
import functools

import jax
import jax.numpy as jnp
from jax import lax
from jax.experimental import pallas as pl
from jax.experimental.pallas import tpu as pltpu
from jax.experimental.pallas import tpu_sc as plsc

N_NODES = 10000
N_PAD = 10240
N_EDGES = 320000
E_PAD = 327680
D_FEAT = 128
D_HID = 16
N_CLASSES = 7

NCORE = 2
NSUB = 16
NW = NCORE * NSUB
EPW = E_PAD // NW
NCHUNK = 80
CW = 128
NB = 4
NGROUP = NCHUNK // NB
RPT = N_PAD // NSUB

@functools.cache
def _get_deg_kernel():
    mesh = plsc.VectorSubcoreMesh(
        core_axis_name="c", subcore_axis_name="s",
        num_cores=NCORE, num_subcores=NSUB)
    return functools.partial(
        pl.kernel,
        out_type=jax.ShapeDtypeStruct((NCORE, N_PAD), jnp.float32),
        mesh=mesh,
        scratch_types=[
            pltpu.VMEM((EPW,), jnp.int32),
            pltpu.VMEM((N_PAD,), jnp.float32),
            pltpu.VMEM((RPT,), jnp.float32),
            pltpu.VMEM((RPT,), jnp.float32),
            pltpu.VMEM_SHARED((NSUB, N_PAD), jnp.float32),
        ],
        compiler_params=pltpu.CompilerParams(needs_layout_passes=False),
    )(_deg_body)


def _deg_body(dst_hbm, out_hbm, dstv, degl, tmp, accv, shared):
    c = lax.axis_index("c")
    s = lax.axis_index("s")
    wid = c * NSUB + s
    pltpu.sync_copy(dst_hbm.at[pl.ds(wid * EPW, EPW)], dstv)
    zeros16 = jnp.zeros((16,), jnp.float32)
    ones16 = jnp.ones((16,), jnp.float32)

    def zero_body(j, carry):
        degl[pl.ds(j * 16, 16)] = zeros16
        return carry
    lax.fori_loop(0, N_PAD // 16, zero_body, 0)

    def count_body(j, carry):
        idx = dstv[pl.ds(j * 16, 16)]
        plsc.addupdate_scatter(degl, [idx], ones16)
        return carry
    lax.fori_loop(0, EPW // 16, count_body, 0)

    pltpu.sync_copy(degl, shared.at[s])
    plsc.subcore_barrier()

    base = s * RPT
    pltpu.sync_copy(shared.at[0, pl.ds(base, RPT)], accv)

    def red_body(t, carry):
        pltpu.sync_copy(shared.at[t, pl.ds(base, RPT)], tmp)

        def add_body(j, carry2):
            accv[pl.ds(j * 16, 16)] = accv[pl.ds(j * 16, 16)] + tmp[pl.ds(j * 16, 16)]
            return carry2
        lax.fori_loop(0, RPT // 16, add_body, 0)
        return carry
    lax.fori_loop(1, NSUB, red_body, 0)

    pltpu.sync_copy(accv, out_hbm.at[c, pl.ds(base, RPT)])


@functools.cache
def _get_edge_kernel():
    mesh = plsc.VectorSubcoreMesh(
        core_axis_name="c", subcore_axis_name="s",
        num_cores=NCORE, num_subcores=NSUB)
    return functools.partial(
        pl.kernel,
        out_type=jax.ShapeDtypeStruct((NCORE, N_PAD, D_HID), jnp.float32),
        mesh=mesh,
        scratch_types=[
            pltpu.VMEM((NCHUNK, CW), jnp.int32),
            pltpu.VMEM((NCHUNK, CW), jnp.int32),
            pltpu.VMEM((2, NB, CW, D_HID), jnp.float32),
            pltpu.VMEM_SHARED((N_PAD, D_HID), jnp.float32),
            pltpu.SemaphoreType.DMA,
            pltpu.SemaphoreType.DMA,
        ],
        compiler_params=pltpu.CompilerParams(
            needs_layout_passes=False, use_tc_tiling_on_sc=False),
    )(_edge_body)


def _edge_body(g_hbm, src_hbm, dst_hbm, zrows_hbm, out_hbm,
               sidx, didx, rows, shared, sem_a, sem_b):
    c = lax.axis_index("c")
    s = lax.axis_index("s")
    wid = c * NSUB + s
    rbase = s * RPT

    pltpu.sync_copy(zrows_hbm, shared.at[pl.ds(rbase, RPT)])
    pltpu.sync_copy(src_hbm.at[wid], sidx)
    pltpu.sync_copy(dst_hbm.at[wid], didx)
    plsc.subcore_barrier()

    def _fire(g, half, sem):
        for b in range(NB):
            pltpu.async_copy(g_hbm.at[sidx.at[g * NB + b]],
                             rows.at[half, b], sem)

    def _drain_scatter(g, half, sem):
        for b in range(NB):
            pltpu.make_async_copy(g_hbm.at[pl.ds(0, CW)],
                                  rows.at[half, b], sem).wait()
            pltpu.sync_copy(rows.at[half, b],
                            shared.at[didx.at[g * NB + b]], add=True)

    _fire(0, 0, sem_a)

    def body(kk, carry):
        g0 = 2 * kk
        _fire(g0 + 1, 1, sem_b)
        _drain_scatter(g0, 0, sem_a)

        @pl.when(g0 + 2 < NGROUP)
        def _():
            _fire(g0 + 2, 0, sem_a)
        _drain_scatter(g0 + 1, 1, sem_b)
        return carry
    lax.fori_loop(0, NGROUP // 2, body, 0)

    plsc.subcore_barrier()
    pltpu.sync_copy(shared.at[pl.ds(rbase, RPT)],
                    out_hbm.at[c, pl.ds(rbase, RPT)])


def _tc_a_body(x_ref, w1_ref, degp_ref, g_ref, dis_ref):
    deg = degp_ref[:, 0:1] + degp_ref[:, 1:2] + 1.0
    dis = lax.rsqrt(deg)
    h = jnp.dot(x_ref[...], w1_ref[...], preferred_element_type=jnp.float32)
    g_ref[...] = h * dis
    dis_ref[...] = dis


_tc_a = pl.pallas_call(
    _tc_a_body,
    out_shape=(jax.ShapeDtypeStruct((N_PAD, D_HID), jnp.float32),
               jax.ShapeDtypeStruct((N_PAD, 1), jnp.float32)),
)


def _tc_b_body(accp_ref, g1_ref, dis_ref, b1_ref, w2_ref, g2_ref):
    acc = accp_ref[0] + accp_ref[1] + g1_ref[...]
    dis = dis_ref[...]
    t = jnp.maximum(acc * dis + b1_ref[...], 0.0)
    g2_ref[...] = jnp.dot(t, w2_ref[...],
                          preferred_element_type=jnp.float32) * dis


_tc_b = pl.pallas_call(
    _tc_b_body,
    out_shape=jax.ShapeDtypeStruct((N_PAD, D_HID), jnp.float32),
)


def _tc_c_body(accp_ref, g2_ref, dis_ref, b2_ref, out_ref):
    acc = accp_ref[0] + accp_ref[1] + g2_ref[...]
    z = acc * dis_ref[...] + b2_ref[...]
    col = lax.broadcasted_iota(jnp.int32, z.shape, 1)
    zm = jnp.where(col < N_CLASSES, z, -jnp.inf)
    m = jnp.max(zm, axis=1, keepdims=True)
    se = jnp.sum(jnp.exp(zm - m), axis=1, keepdims=True)
    out_ref[...] = z - m - jnp.log(se)


_tc_c = pl.pallas_call(
    _tc_c_body,
    out_shape=jax.ShapeDtypeStruct((N_PAD, D_HID), jnp.float32),
)


def kernel(x, edge_index, W1, b1, W2, b2):
    src = edge_index[0].astype(jnp.int32)
    dst = edge_index[1].astype(jnp.int32)
    padv = jnp.full((E_PAD - N_EDGES,), N_NODES, jnp.int32)
    src_flat = jnp.concatenate([src, padv])
    dst_flat = jnp.concatenate([dst, padv])
    src3 = src_flat.reshape(NW, NCHUNK, CW)
    dst3 = dst_flat.reshape(NW, NCHUNK, CW)

    xp = jnp.pad(x, ((0, N_PAD - N_NODES), (0, 0)))
    w2p = jnp.pad(W2, ((0, 0), (0, D_HID - N_CLASSES)))
    b1r = b1.reshape(1, D_HID)
    b2r = jnp.pad(b2, (0, D_HID - N_CLASSES)).reshape(1, D_HID)
    zrows = jnp.zeros((RPT, D_HID), jnp.float32)

    deg_kernel = _get_deg_kernel()
    edge_kernel = _get_edge_kernel()
    degp = deg_kernel(dst_flat)
    g1, dis = _tc_a(xp, W1, degp.T)
    accp1 = edge_kernel(g1, src3, dst3, zrows)
    g2 = _tc_b(accp1, g1, dis, b1r, w2p)
    accp2 = edge_kernel(g2, src3, dst3, zrows)
    z = _tc_c(accp2, g2, dis, b2r)
    return z[:N_NODES, :N_CLASSES]


# speedup vs baseline: 1.3378x; 1.3281x over previous
import functools

import jax
import jax.numpy as jnp
from jax import lax
from jax.experimental import pallas as pl
from jax.experimental.pallas import tpu as pltpu
from jax.experimental.pallas import tpu_sc as plsc

N_NODES = 10000
N_PAD = 10240
N_EDGES = 320000
E_PAD = 327680
D_FEAT = 128
D_HID = 16
N_CLASSES = 7

NCORE = 2
NSUB = 16
NW = NCORE * NSUB
EPW = E_PAD // NW
NCHUNK = 80
CW = 128
NB = 4
NGROUP = NCHUNK // NB
RPT = N_PAD // NSUB

@functools.cache
def _get_deg_kernel():
    mesh = plsc.VectorSubcoreMesh(
        core_axis_name="c", subcore_axis_name="s",
        num_cores=NCORE, num_subcores=NSUB)
    return functools.partial(
        pl.kernel,
        out_type=jax.ShapeDtypeStruct((NCORE, N_PAD), jnp.float32),
        mesh=mesh,
        scratch_types=[
            pltpu.VMEM((EPW,), jnp.int32),
            pltpu.VMEM((N_PAD,), jnp.float32),
            pltpu.VMEM((RPT,), jnp.float32),
            pltpu.VMEM((RPT,), jnp.float32),
            pltpu.VMEM_SHARED((NSUB, N_PAD), jnp.float32),
        ],
        compiler_params=pltpu.CompilerParams(needs_layout_passes=False),
    )(_deg_body)


def _deg_body(dst_hbm, out_hbm, dstv, degl, tmp, accv, shared):
    c = lax.axis_index("c")
    s = lax.axis_index("s")
    wid = c * NSUB + s
    pltpu.sync_copy(dst_hbm.at[pl.ds(wid * EPW, EPW)], dstv)
    zeros16 = jnp.zeros((16,), jnp.float32)
    ones16 = jnp.ones((16,), jnp.float32)

    def zero_body(j, carry):
        degl[pl.ds(j * 16, 16)] = zeros16
        return carry
    lax.fori_loop(0, N_PAD // 16, zero_body, 0)

    def count_body(j, carry):
        idx = dstv[pl.ds(j * 16, 16)]
        plsc.addupdate_scatter(degl, [idx], ones16)
        return carry
    lax.fori_loop(0, EPW // 16, count_body, 0)

    pltpu.sync_copy(degl, shared.at[s])
    plsc.subcore_barrier()

    base = s * RPT
    pltpu.sync_copy(shared.at[0, pl.ds(base, RPT)], accv)

    def red_body(t, carry):
        pltpu.sync_copy(shared.at[t, pl.ds(base, RPT)], tmp)

        def add_body(j, carry2):
            accv[pl.ds(j * 16, 16)] = accv[pl.ds(j * 16, 16)] + tmp[pl.ds(j * 16, 16)]
            return carry2
        lax.fori_loop(0, RPT // 16, add_body, 0)
        return carry
    lax.fori_loop(1, NSUB, red_body, 0)

    pltpu.sync_copy(accv, out_hbm.at[c, pl.ds(base, RPT)])


@functools.cache
def _get_edge_kernel():
    mesh = plsc.VectorSubcoreMesh(
        core_axis_name="c", subcore_axis_name="s",
        num_cores=NCORE, num_subcores=NSUB)
    return functools.partial(
        pl.kernel,
        out_type=jax.ShapeDtypeStruct((NCORE, N_PAD, D_HID), jnp.float32),
        mesh=mesh,
        scratch_types=[
            pltpu.VMEM((NCHUNK, CW), jnp.int32),
            pltpu.VMEM((NCHUNK, CW), jnp.int32),
            pltpu.VMEM((2, NB, CW, D_HID), jnp.float32),
            pltpu.VMEM_SHARED((N_PAD, D_HID), jnp.float32),
            pltpu.VMEM_SHARED((N_PAD, D_HID), jnp.float32),
            pltpu.SemaphoreType.DMA,
            pltpu.SemaphoreType.DMA,
        ],
        compiler_params=pltpu.CompilerParams(
            needs_layout_passes=False, use_tc_tiling_on_sc=False),
    )(_edge_body)


def _edge_body(g_hbm, src_hbm, dst_hbm, zrows_hbm, out_hbm,
               sidx, didx, rows, shared, shared_g, sem_a, sem_b):
    c = lax.axis_index("c")
    s = lax.axis_index("s")
    wid = c * NSUB + s
    rbase = s * RPT

    pltpu.sync_copy(zrows_hbm, shared.at[pl.ds(rbase, RPT)])
    pltpu.sync_copy(g_hbm.at[pl.ds(rbase, RPT)], shared_g.at[pl.ds(rbase, RPT)])
    pltpu.sync_copy(src_hbm.at[wid], sidx)
    pltpu.sync_copy(dst_hbm.at[wid], didx)
    plsc.subcore_barrier()

    def _fire(g, half, sem):
        for b in range(NB):
            pltpu.async_copy(shared_g.at[sidx.at[g * NB + b]],
                             rows.at[half, b], sem)

    def _drain_scatter(g, half, sem):
        for b in range(NB):
            pltpu.make_async_copy(g_hbm.at[pl.ds(0, CW)],
                                  rows.at[half, b], sem).wait()
            pltpu.sync_copy(rows.at[half, b],
                            shared.at[didx.at[g * NB + b]], add=True)

    _fire(0, 0, sem_a)

    def body(kk, carry):
        g0 = 2 * kk
        _fire(g0 + 1, 1, sem_b)
        _drain_scatter(g0, 0, sem_a)

        @pl.when(g0 + 2 < NGROUP)
        def _():
            _fire(g0 + 2, 0, sem_a)
        _drain_scatter(g0 + 1, 1, sem_b)
        return carry
    lax.fori_loop(0, NGROUP // 2, body, 0)

    plsc.subcore_barrier()
    pltpu.sync_copy(shared.at[pl.ds(rbase, RPT)],
                    out_hbm.at[c, pl.ds(rbase, RPT)])


def _tc_a_body(x_ref, w1_ref, degp_ref, g_ref, dis_ref):
    deg = degp_ref[:, 0:1] + degp_ref[:, 1:2] + 1.0
    dis = lax.rsqrt(deg)
    h = jnp.dot(x_ref[...], w1_ref[...], preferred_element_type=jnp.float32)
    g_ref[...] = h * dis
    dis_ref[...] = dis


_tc_a = pl.pallas_call(
    _tc_a_body,
    out_shape=(jax.ShapeDtypeStruct((N_PAD, D_HID), jnp.float32),
               jax.ShapeDtypeStruct((N_PAD, 1), jnp.float32)),
)


def _tc_b_body(accp_ref, g1_ref, dis_ref, b1_ref, w2_ref, g2_ref):
    acc = accp_ref[0] + accp_ref[1] + g1_ref[...]
    dis = dis_ref[...]
    t = jnp.maximum(acc * dis + b1_ref[...], 0.0)
    g2_ref[...] = jnp.dot(t, w2_ref[...],
                          preferred_element_type=jnp.float32) * dis


_tc_b = pl.pallas_call(
    _tc_b_body,
    out_shape=jax.ShapeDtypeStruct((N_PAD, D_HID), jnp.float32),
)


def _tc_c_body(accp_ref, g2_ref, dis_ref, b2_ref, out_ref):
    acc = accp_ref[0] + accp_ref[1] + g2_ref[...]
    z = acc * dis_ref[...] + b2_ref[...]
    col = lax.broadcasted_iota(jnp.int32, z.shape, 1)
    zm = jnp.where(col < N_CLASSES, z, -jnp.inf)
    m = jnp.max(zm, axis=1, keepdims=True)
    se = jnp.sum(jnp.exp(zm - m), axis=1, keepdims=True)
    out_ref[...] = z - m - jnp.log(se)


_tc_c = pl.pallas_call(
    _tc_c_body,
    out_shape=jax.ShapeDtypeStruct((N_PAD, D_HID), jnp.float32),
)


def kernel(x, edge_index, W1, b1, W2, b2):
    src = edge_index[0].astype(jnp.int32)
    dst = edge_index[1].astype(jnp.int32)
    padv = jnp.full((E_PAD - N_EDGES,), N_NODES, jnp.int32)
    src_flat = jnp.concatenate([src, padv])
    dst_flat = jnp.concatenate([dst, padv])
    src3 = src_flat.reshape(NW, NCHUNK, CW)
    dst3 = dst_flat.reshape(NW, NCHUNK, CW)

    xp = jnp.pad(x, ((0, N_PAD - N_NODES), (0, 0)))
    w2p = jnp.pad(W2, ((0, 0), (0, D_HID - N_CLASSES)))
    b1r = b1.reshape(1, D_HID)
    b2r = jnp.pad(b2, (0, D_HID - N_CLASSES)).reshape(1, D_HID)
    zrows = jnp.zeros((RPT, D_HID), jnp.float32)

    deg_kernel = _get_deg_kernel()
    edge_kernel = _get_edge_kernel()
    degp = deg_kernel(dst_flat)
    g1, dis = _tc_a(xp, W1, degp.T)
    accp1 = edge_kernel(g1, src3, dst3, zrows)
    g2 = _tc_b(accp1, g1, dis, b1r, w2p)
    accp2 = edge_kernel(g2, src3, dst3, zrows)
    z = _tc_c(accp2, g2, dis, b2r)
    return z[:N_NODES, :N_CLASSES]


# speedup vs baseline: 1.3560x; 1.0136x over previous
import functools

import jax
import jax.numpy as jnp
from jax import lax
from jax.experimental import pallas as pl
from jax.experimental.pallas import tpu as pltpu
from jax.experimental.pallas import tpu_sc as plsc

N_NODES = 10000
N_PAD = 10240
N_EDGES = 320000
E_PAD = 327680
D_FEAT = 128
D_HID = 16
N_CLASSES = 7

NCORE = 2
NSUB = 16
NW = NCORE * NSUB
EPW = E_PAD // NW
EPT = E_PAD // NSUB
NCHUNK = 80
CW = 128
NB = 4
NGROUP = NCHUNK // NB
RPT = N_PAD // NSUB


def _edge_pipeline(shared_g, shared, sidx, didx, rows, sem_a, sem_b):
    def _fire(g, half, sem):
        for b in range(NB):
            pltpu.async_copy(shared_g.at[sidx.at[g * NB + b]],
                             rows.at[half, b], sem)

    def _drain_scatter(g, half, sem):
        for b in range(NB):
            pltpu.make_async_copy(shared_g.at[pl.ds(0, CW)],
                                  rows.at[half, b], sem).wait()
            pltpu.sync_copy(rows.at[half, b],
                            shared.at[didx.at[g * NB + b]], add=True)

    _fire(0, 0, sem_a)

    def body(kk, carry):
        g0 = 2 * kk
        _fire(g0 + 1, 1, sem_b)
        _drain_scatter(g0, 0, sem_a)

        @pl.when(g0 + 2 < NGROUP)
        def _():
            _fire(g0 + 2, 0, sem_a)
        _drain_scatter(g0 + 1, 1, sem_b)
        return carry
    lax.fori_loop(0, NGROUP // 2, body, 0)


def _rsqrt16(v):
    xi = lax.bitcast_convert_type(v, jnp.int32)
    yi = jnp.int32(0x5F3759DF) - (xi >> 1)
    y = lax.bitcast_convert_type(yi, jnp.float32)
    for _ in range(3):
        y = y * (1.5 - 0.5 * v * y * y)
    return y


@functools.cache
def _get_l1_kernel():
    mesh = plsc.VectorSubcoreMesh(
        core_axis_name="c", subcore_axis_name="s",
        num_cores=NCORE, num_subcores=NSUB)
    return functools.partial(
        pl.kernel,
        out_type=(jax.ShapeDtypeStruct((NCORE, N_PAD, D_HID), jnp.float32),
                  jax.ShapeDtypeStruct((N_PAD,), jnp.float32)),
        mesh=mesh,
        scratch_types=[
            pltpu.VMEM((NCHUNK, CW), jnp.int32),
            pltpu.VMEM((NCHUNK, CW), jnp.int32),
            pltpu.VMEM((2, NB, CW, D_HID), jnp.float32),
            pltpu.VMEM((2, NCHUNK, CW), jnp.int32),
            pltpu.VMEM((N_PAD,), jnp.float32),
            pltpu.VMEM((NSUB, RPT), jnp.float32),
            pltpu.VMEM((RPT,), jnp.float32),
            pltpu.VMEM((RPT, D_HID), jnp.float32),
            pltpu.VMEM_SHARED((N_PAD, D_HID), jnp.float32),
            pltpu.VMEM_SHARED((N_PAD, D_HID), jnp.float32),
            pltpu.VMEM_SHARED((NSUB, N_PAD), jnp.float32),
            pltpu.SemaphoreType.DMA,
            pltpu.SemaphoreType.DMA,
        ],
        compiler_params=pltpu.CompilerParams(
            needs_layout_passes=False, use_tc_tiling_on_sc=False),
    )(_l1_body)


def _l1_body(h_hbm, src_hbm, dst_hbm, zrows_hbm,
             out_hbm, dis_hbm,
             sidx, didx, rows, dstv, hist, red, disv, hbuf,
             shared, shared_g, shared_p, sem_a, sem_b):
    c = lax.axis_index("c")
    s = lax.axis_index("s")
    wid = c * NSUB + s
    rbase = s * RPT

    pltpu.sync_copy(zrows_hbm, shared.at[pl.ds(rbase, RPT)])
    pltpu.sync_copy(h_hbm.at[pl.ds(rbase, RPT)], hbuf)
    pltpu.sync_copy(dst_hbm.at[2 * s], dstv.at[0])
    pltpu.sync_copy(dst_hbm.at[2 * s + 1], dstv.at[1])
    pltpu.sync_copy(src_hbm.at[wid], sidx)
    pltpu.sync_copy(dst_hbm.at[wid], didx)

    zeros16 = jnp.zeros((16,), jnp.float32)
    ones16 = jnp.ones((16,), jnp.float32)

    def zero_body(j, carry):
        hist[pl.ds(j * 16, 16)] = zeros16
        return carry
    lax.fori_loop(0, N_PAD // 16, zero_body, 0)

    def count_body(m, carry):
        w = m // NCHUNK
        ch = m % NCHUNK
        for k in range(CW // 16):
            idx = dstv[w, ch, pl.ds(k * 16, 16)]
            plsc.addupdate_scatter(hist, [idx], ones16)
        return carry
    lax.fori_loop(0, 2 * NCHUNK, count_body, 0)

    pltpu.sync_copy(hist, shared_p.at[s])
    plsc.subcore_barrier()

    pltpu.sync_copy(shared_p.at[:, pl.ds(rbase, RPT)], red)

    def dis_body(jj, carry):
        col = jj * 16
        v = red[0, pl.ds(col, 16)]
        for t in range(1, NSUB):
            v = v + red[t, pl.ds(col, 16)]
        disv[pl.ds(col, 16)] = _rsqrt16(v + 1.0)
        return carry
    lax.fori_loop(0, RPT // 16, dis_body, 0)

    @pl.when(c == 0)
    def _():
        pltpu.sync_copy(disv, dis_hbm.at[pl.ds(rbase, RPT)])

    def scale_body(jj, carry):
        col = jj * 16
        dv = disv[pl.ds(col, 16)]
        for i in range(16):
            hbuf[col + i] = hbuf[col + i] * dv[i]
        return carry
    lax.fori_loop(0, RPT // 16, scale_body, 0)
    pltpu.sync_copy(hbuf, shared_g.at[pl.ds(rbase, RPT)])
    plsc.subcore_barrier()

    _edge_pipeline(shared_g, shared, sidx, didx, rows, sem_a, sem_b)

    plsc.subcore_barrier()
    pltpu.sync_copy(shared.at[pl.ds(rbase, RPT)],
                    out_hbm.at[c, pl.ds(rbase, RPT)])


@functools.cache
def _get_edge_kernel():
    mesh = plsc.VectorSubcoreMesh(
        core_axis_name="c", subcore_axis_name="s",
        num_cores=NCORE, num_subcores=NSUB)
    return functools.partial(
        pl.kernel,
        out_type=jax.ShapeDtypeStruct((NCORE, N_PAD, D_HID), jnp.float32),
        mesh=mesh,
        scratch_types=[
            pltpu.VMEM((NCHUNK, CW), jnp.int32),
            pltpu.VMEM((NCHUNK, CW), jnp.int32),
            pltpu.VMEM((2, NB, CW, D_HID), jnp.float32),
            pltpu.VMEM_SHARED((N_PAD, D_HID), jnp.float32),
            pltpu.VMEM_SHARED((N_PAD, D_HID), jnp.float32),
            pltpu.SemaphoreType.DMA,
            pltpu.SemaphoreType.DMA,
        ],
        compiler_params=pltpu.CompilerParams(
            needs_layout_passes=False, use_tc_tiling_on_sc=False),
    )(_edge_body)


def _edge_body(g_hbm, src_hbm, dst_hbm, zrows_hbm, out_hbm,
               sidx, didx, rows, shared, shared_g, sem_a, sem_b):
    c = lax.axis_index("c")
    s = lax.axis_index("s")
    wid = c * NSUB + s
    rbase = s * RPT

    pltpu.sync_copy(zrows_hbm, shared.at[pl.ds(rbase, RPT)])
    pltpu.sync_copy(g_hbm.at[pl.ds(rbase, RPT)], shared_g.at[pl.ds(rbase, RPT)])
    pltpu.sync_copy(src_hbm.at[wid], sidx)
    pltpu.sync_copy(dst_hbm.at[wid], didx)
    plsc.subcore_barrier()

    _edge_pipeline(shared_g, shared, sidx, didx, rows, sem_a, sem_b)

    plsc.subcore_barrier()
    pltpu.sync_copy(shared.at[pl.ds(rbase, RPT)],
                    out_hbm.at[c, pl.ds(rbase, RPT)])


def _tc_m_body(x_ref, w1_ref, h_ref):
    h_ref[...] = jnp.dot(x_ref[...], w1_ref[...],
                         preferred_element_type=jnp.float32)


_tc_m = pl.pallas_call(
    _tc_m_body,
    out_shape=jax.ShapeDtypeStruct((N_PAD, D_HID), jnp.float32),
)


def _tc_b_body(accp_ref, h1_ref, dis_ref, b1_ref, w2_ref, g2_ref):
    dis = dis_ref[...]
    acc = accp_ref[0] + accp_ref[1] + h1_ref[...] * dis
    t = jnp.maximum(acc * dis + b1_ref[...], 0.0)
    g2_ref[...] = jnp.dot(t, w2_ref[...],
                          preferred_element_type=jnp.float32) * dis


_tc_b = pl.pallas_call(
    _tc_b_body,
    out_shape=jax.ShapeDtypeStruct((N_PAD, D_HID), jnp.float32),
)


def _tc_c_body(accp_ref, g2_ref, dis_ref, b2_ref, out_ref):
    acc = accp_ref[0] + accp_ref[1] + g2_ref[...]
    z = acc * dis_ref[...] + b2_ref[...]
    col = lax.broadcasted_iota(jnp.int32, z.shape, 1)
    zm = jnp.where(col < N_CLASSES, z, -jnp.inf)
    m = jnp.max(zm, axis=1, keepdims=True)
    se = jnp.sum(jnp.exp(zm - m), axis=1, keepdims=True)
    out_ref[...] = z - m - jnp.log(se)


_tc_c = pl.pallas_call(
    _tc_c_body,
    out_shape=jax.ShapeDtypeStruct((N_PAD, D_HID), jnp.float32),
)


def kernel(x, edge_index, W1, b1, W2, b2):
    src = edge_index[0].astype(jnp.int32)
    dst = edge_index[1].astype(jnp.int32)
    padv = jnp.full((E_PAD - N_EDGES,), N_NODES, jnp.int32)
    src3 = jnp.concatenate([src, padv]).reshape(NW, NCHUNK, CW)
    dst3 = jnp.concatenate([dst, padv]).reshape(NW, NCHUNK, CW)

    xp = jnp.pad(x, ((0, N_PAD - N_NODES), (0, 0)))
    w2p = jnp.pad(W2, ((0, 0), (0, D_HID - N_CLASSES)))
    b1r = b1.reshape(1, D_HID)
    b2r = jnp.pad(b2, (0, D_HID - N_CLASSES)).reshape(1, D_HID)
    zrows = jnp.zeros((RPT, D_HID), jnp.float32)

    h1 = _tc_m(xp, W1)
    accp1, dis = _get_l1_kernel()(h1, src3, dst3, zrows)
    dis2 = dis.reshape(N_PAD, 1)
    g2 = _tc_b(accp1, h1, dis2, b1r, w2p)
    accp2 = _get_edge_kernel()(g2, src3, dst3, zrows)
    z = _tc_c(accp2, g2, dis2, b2r)
    return z[:N_NODES, :N_CLASSES]


# speedup vs baseline: 1.3577x; 1.0012x over previous
import functools

import jax
import jax.numpy as jnp
from jax import lax
from jax.experimental import pallas as pl
from jax.experimental.pallas import tpu as pltpu
from jax.experimental.pallas import tpu_sc as plsc

N_NODES = 10000
N_PAD = 10240
N_EDGES = 320000
E_PAD = 327680
D_FEAT = 128
D_HID = 16
N_CLASSES = 7

NCORE = 2
NSUB = 16
NW = NCORE * NSUB
EPW = E_PAD // NW
EPT = E_PAD // NSUB
NCHUNK = 80
CW = 128
NB = 4
NGROUP = NCHUNK // NB
RPT = N_PAD // NSUB


def _edge_pipeline(shared_g, shared, sidx, didx, rows, sem_a, sem_b, sem_c):
    def _fire(g, half, sem):
        for b in range(NB):
            pltpu.async_copy(shared_g.at[sidx.at[g * NB + b]],
                             rows.at[half, b], sem)

    def _drain_scatter(g, half, sem):
        for b in range(NB):
            pltpu.make_async_copy(shared_g.at[pl.ds(0, CW)],
                                  rows.at[half, b], sem).wait()
        for b in range(NB):
            pltpu.async_copy(rows.at[half, b],
                             shared.at[didx.at[g * NB + b]], sem_c, add=True)
        for b in range(NB):
            pltpu.make_async_copy(shared_g.at[pl.ds(0, CW)],
                                  rows.at[half, b], sem_c).wait()

    _fire(0, 0, sem_a)

    def body(kk, carry):
        g0 = 2 * kk
        _fire(g0 + 1, 1, sem_b)
        _drain_scatter(g0, 0, sem_a)

        @pl.when(g0 + 2 < NGROUP)
        def _():
            _fire(g0 + 2, 0, sem_a)
        _drain_scatter(g0 + 1, 1, sem_b)
        return carry
    lax.fori_loop(0, NGROUP // 2, body, 0)


def _rsqrt16(v):
    xi = lax.bitcast_convert_type(v, jnp.int32)
    yi = jnp.int32(0x5F3759DF) - (xi >> 1)
    y = lax.bitcast_convert_type(yi, jnp.float32)
    for _ in range(3):
        y = y * (1.5 - 0.5 * v * y * y)
    return y


@functools.cache
def _get_l1_kernel():
    mesh = plsc.VectorSubcoreMesh(
        core_axis_name="c", subcore_axis_name="s",
        num_cores=NCORE, num_subcores=NSUB)
    return functools.partial(
        pl.kernel,
        out_type=(jax.ShapeDtypeStruct((NCORE, N_PAD, D_HID), jnp.float32),
                  jax.ShapeDtypeStruct((N_PAD,), jnp.float32)),
        mesh=mesh,
        scratch_types=[
            pltpu.VMEM((NCHUNK, CW), jnp.int32),
            pltpu.VMEM((NCHUNK, CW), jnp.int32),
            pltpu.VMEM((2, NB, CW, D_HID), jnp.float32),
            pltpu.VMEM((2, NCHUNK, CW), jnp.int32),
            pltpu.VMEM((N_PAD,), jnp.float32),
            pltpu.VMEM((NSUB, RPT), jnp.float32),
            pltpu.VMEM((RPT,), jnp.float32),
            pltpu.VMEM((RPT, D_HID), jnp.float32),
            pltpu.VMEM_SHARED((N_PAD, D_HID), jnp.float32),
            pltpu.VMEM_SHARED((N_PAD, D_HID), jnp.float32),
            pltpu.VMEM_SHARED((NSUB, N_PAD), jnp.float32),
            pltpu.SemaphoreType.DMA,
            pltpu.SemaphoreType.DMA,
            pltpu.SemaphoreType.DMA,
        ],
        compiler_params=pltpu.CompilerParams(
            needs_layout_passes=False, use_tc_tiling_on_sc=False),
    )(_l1_body)


def _l1_body(h_hbm, src_hbm, dst_hbm, zrows_hbm,
             out_hbm, dis_hbm,
             sidx, didx, rows, dstv, hist, red, disv, hbuf,
             shared, shared_g, shared_p, sem_a, sem_b, sem_c):
    c = lax.axis_index("c")
    s = lax.axis_index("s")
    wid = c * NSUB + s
    rbase = s * RPT

    pltpu.sync_copy(zrows_hbm, shared.at[pl.ds(rbase, RPT)])
    pltpu.sync_copy(h_hbm.at[pl.ds(rbase, RPT)], hbuf)
    pltpu.sync_copy(dst_hbm.at[2 * s], dstv.at[0])
    pltpu.sync_copy(dst_hbm.at[2 * s + 1], dstv.at[1])
    pltpu.sync_copy(src_hbm.at[wid], sidx)
    pltpu.sync_copy(dst_hbm.at[wid], didx)

    zeros16 = jnp.zeros((16,), jnp.float32)
    ones16 = jnp.ones((16,), jnp.float32)

    def zero_body(j, carry):
        hist[pl.ds(j * 16, 16)] = zeros16
        return carry
    lax.fori_loop(0, N_PAD // 16, zero_body, 0)

    def count_body(m, carry):
        w = m // NCHUNK
        ch = m % NCHUNK
        for k in range(CW // 16):
            idx = dstv[w, ch, pl.ds(k * 16, 16)]
            plsc.addupdate_scatter(hist, [idx], ones16)
        return carry
    lax.fori_loop(0, 2 * NCHUNK, count_body, 0)

    pltpu.sync_copy(hist, shared_p.at[s])
    plsc.subcore_barrier()

    pltpu.sync_copy(shared_p.at[:, pl.ds(rbase, RPT)], red)

    def dis_body(jj, carry):
        col = jj * 16
        v = red[0, pl.ds(col, 16)]
        for t in range(1, NSUB):
            v = v + red[t, pl.ds(col, 16)]
        disv[pl.ds(col, 16)] = _rsqrt16(v + 1.0)
        return carry
    lax.fori_loop(0, RPT // 16, dis_body, 0)

    @pl.when(c == 0)
    def _():
        pltpu.sync_copy(disv, dis_hbm.at[pl.ds(rbase, RPT)])

    def scale_body(jj, carry):
        col = jj * 16
        dv = disv[pl.ds(col, 16)]
        for i in range(16):
            hbuf[col + i] = hbuf[col + i] * dv[i]
        return carry
    lax.fori_loop(0, RPT // 16, scale_body, 0)
    pltpu.sync_copy(hbuf, shared_g.at[pl.ds(rbase, RPT)])
    plsc.subcore_barrier()

    _edge_pipeline(shared_g, shared, sidx, didx, rows, sem_a, sem_b, sem_c)

    plsc.subcore_barrier()
    pltpu.sync_copy(shared.at[pl.ds(rbase, RPT)],
                    out_hbm.at[c, pl.ds(rbase, RPT)])


@functools.cache
def _get_edge_kernel():
    mesh = plsc.VectorSubcoreMesh(
        core_axis_name="c", subcore_axis_name="s",
        num_cores=NCORE, num_subcores=NSUB)
    return functools.partial(
        pl.kernel,
        out_type=jax.ShapeDtypeStruct((NCORE, N_PAD, D_HID), jnp.float32),
        mesh=mesh,
        scratch_types=[
            pltpu.VMEM((NCHUNK, CW), jnp.int32),
            pltpu.VMEM((NCHUNK, CW), jnp.int32),
            pltpu.VMEM((2, NB, CW, D_HID), jnp.float32),
            pltpu.VMEM_SHARED((N_PAD, D_HID), jnp.float32),
            pltpu.VMEM_SHARED((N_PAD, D_HID), jnp.float32),
            pltpu.SemaphoreType.DMA,
            pltpu.SemaphoreType.DMA,
            pltpu.SemaphoreType.DMA,
        ],
        compiler_params=pltpu.CompilerParams(
            needs_layout_passes=False, use_tc_tiling_on_sc=False),
    )(_edge_body)


def _edge_body(g_hbm, src_hbm, dst_hbm, zrows_hbm, out_hbm,
               sidx, didx, rows, shared, shared_g, sem_a, sem_b, sem_c):
    c = lax.axis_index("c")
    s = lax.axis_index("s")
    wid = c * NSUB + s
    rbase = s * RPT

    pltpu.sync_copy(zrows_hbm, shared.at[pl.ds(rbase, RPT)])
    pltpu.sync_copy(g_hbm.at[pl.ds(rbase, RPT)], shared_g.at[pl.ds(rbase, RPT)])
    pltpu.sync_copy(src_hbm.at[wid], sidx)
    pltpu.sync_copy(dst_hbm.at[wid], didx)
    plsc.subcore_barrier()

    _edge_pipeline(shared_g, shared, sidx, didx, rows, sem_a, sem_b, sem_c)

    plsc.subcore_barrier()
    pltpu.sync_copy(shared.at[pl.ds(rbase, RPT)],
                    out_hbm.at[c, pl.ds(rbase, RPT)])


def _tc_m_body(x_ref, w1_ref, h_ref):
    h_ref[...] = jnp.dot(x_ref[...], w1_ref[...],
                         preferred_element_type=jnp.float32)


_tc_m = pl.pallas_call(
    _tc_m_body,
    out_shape=jax.ShapeDtypeStruct((N_PAD, D_HID), jnp.float32),
)


def _tc_b_body(accp_ref, h1_ref, dis_ref, b1_ref, w2_ref, g2_ref):
    dis = dis_ref[...]
    acc = accp_ref[0] + accp_ref[1] + h1_ref[...] * dis
    t = jnp.maximum(acc * dis + b1_ref[...], 0.0)
    g2_ref[...] = jnp.dot(t, w2_ref[...],
                          preferred_element_type=jnp.float32) * dis


_tc_b = pl.pallas_call(
    _tc_b_body,
    out_shape=jax.ShapeDtypeStruct((N_PAD, D_HID), jnp.float32),
)


def _tc_c_body(accp_ref, g2_ref, dis_ref, b2_ref, out_ref):
    acc = accp_ref[0] + accp_ref[1] + g2_ref[...]
    z = acc * dis_ref[...] + b2_ref[...]
    col = lax.broadcasted_iota(jnp.int32, z.shape, 1)
    zm = jnp.where(col < N_CLASSES, z, -jnp.inf)
    m = jnp.max(zm, axis=1, keepdims=True)
    se = jnp.sum(jnp.exp(zm - m), axis=1, keepdims=True)
    out_ref[...] = z - m - jnp.log(se)


_tc_c = pl.pallas_call(
    _tc_c_body,
    out_shape=jax.ShapeDtypeStruct((N_PAD, D_HID), jnp.float32),
)


def kernel(x, edge_index, W1, b1, W2, b2):
    src = edge_index[0].astype(jnp.int32)
    dst = edge_index[1].astype(jnp.int32)
    padv = jnp.full((E_PAD - N_EDGES,), N_NODES, jnp.int32)
    src3 = jnp.concatenate([src, padv]).reshape(NW, NCHUNK, CW)
    dst3 = jnp.concatenate([dst, padv]).reshape(NW, NCHUNK, CW)

    xp = jnp.pad(x, ((0, N_PAD - N_NODES), (0, 0)))
    w2p = jnp.pad(W2, ((0, 0), (0, D_HID - N_CLASSES)))
    b1r = b1.reshape(1, D_HID)
    b2r = jnp.pad(b2, (0, D_HID - N_CLASSES)).reshape(1, D_HID)
    zrows = jnp.zeros((RPT, D_HID), jnp.float32)

    h1 = _tc_m(xp, W1)
    accp1, dis = _get_l1_kernel()(h1, src3, dst3, zrows)
    dis2 = dis.reshape(N_PAD, 1)
    g2 = _tc_b(accp1, h1, dis2, b1r, w2p)
    accp2 = _get_edge_kernel()(g2, src3, dst3, zrows)
    z = _tc_c(accp2, g2, dis2, b2r)
    return z[:N_NODES, :N_CLASSES]


# speedup vs baseline: 1.8764x; 1.3821x over previous
import functools

import jax
import jax.numpy as jnp
from jax import lax
from jax.experimental import pallas as pl
from jax.experimental.pallas import tpu as pltpu
from jax.experimental.pallas import tpu_sc as plsc

N_NODES = 10000
N_PAD = 10240
N_EDGES = 320000
E_PAD = 327680
D_FEAT = 128
D_HID = 16
N_CLASSES = 7

NCORE = 2
NSUB = 16
NW = NCORE * NSUB
EPW = E_PAD // NW
NCHUNK = 80
CW = 128
NB = 4
NGROUP = NCHUNK // NB
RPT = N_PAD // NSUB
NR = N_PAD * D_HID // 128


def _edge_pipeline(shared_g, shared, sidx, didx, rows, sem_a, sem_b, sem_c):
    def _fire(g, half, sem):
        for b in range(NB):
            pltpu.async_copy(shared_g.at[sidx.at[g * NB + b]],
                             rows.at[half, b], sem)

    def _drain_scatter(g, half, sem):
        for b in range(NB):
            pltpu.make_async_copy(shared_g.at[pl.ds(0, CW)],
                                  rows.at[half, b], sem).wait()
        for b in range(NB):
            pltpu.async_copy(rows.at[half, b],
                             shared.at[didx.at[g * NB + b]], sem_c, add=True)
        for b in range(NB):
            pltpu.make_async_copy(shared_g.at[pl.ds(0, CW)],
                                  rows.at[half, b], sem_c).wait()

    _fire(0, 0, sem_a)

    def body(kk, carry):
        g0 = 2 * kk
        _fire(g0 + 1, 1, sem_b)
        _drain_scatter(g0, 0, sem_a)

        @pl.when(g0 + 2 < NGROUP)
        def _():
            _fire(g0 + 2, 0, sem_a)
        _drain_scatter(g0 + 1, 1, sem_b)
        return carry
    lax.fori_loop(0, NGROUP // 2, body, 0)


def _rsqrt16(v):
    xi = lax.bitcast_convert_type(v, jnp.int32)
    yi = jnp.int32(0x5F3759DF) - (xi >> 1)
    y = lax.bitcast_convert_type(yi, jnp.float32)
    for _ in range(3):
        y = y * (1.5 - 0.5 * v * y * y)
    return y


@functools.cache
def _get_l1_kernel():
    mesh = plsc.VectorSubcoreMesh(
        core_axis_name="c", subcore_axis_name="s",
        num_cores=NCORE, num_subcores=NSUB)
    return functools.partial(
        pl.kernel,
        out_type=(jax.ShapeDtypeStruct((NCORE, N_PAD, D_HID), jnp.float32),
                  jax.ShapeDtypeStruct((N_PAD, D_HID), jnp.float32)),
        mesh=mesh,
        scratch_types=[
            pltpu.VMEM((NCHUNK, CW), jnp.int32),
            pltpu.VMEM((NCHUNK, CW), jnp.int32),
            pltpu.VMEM((2, NB, CW, D_HID), jnp.float32),
            pltpu.VMEM((2, NCHUNK, CW), jnp.int32),
            pltpu.VMEM((N_PAD,), jnp.float32),
            pltpu.VMEM((NSUB, RPT), jnp.float32),
            pltpu.VMEM((RPT,), jnp.float32),
            pltpu.VMEM((RPT, D_HID), jnp.float32),
            pltpu.VMEM((RPT, D_HID), jnp.float32),
            pltpu.VMEM_SHARED((N_PAD, D_HID), jnp.float32),
            pltpu.VMEM_SHARED((N_PAD, D_HID), jnp.float32),
            pltpu.VMEM_SHARED((NSUB, N_PAD), jnp.float32),
            pltpu.SemaphoreType.DMA,
            pltpu.SemaphoreType.DMA,
            pltpu.SemaphoreType.DMA,
        ],
        compiler_params=pltpu.CompilerParams(
            needs_layout_passes=False, use_tc_tiling_on_sc=False),
    )(_l1_body)


def _l1_body(h_hbm, ei_hbm, zrows_hbm,
             out_hbm, dis_hbm,
             sidx, didx, rows, dstv, hist, red, disv, hbuf, disb,
             shared, shared_g, shared_p, sem_a, sem_b, sem_c):
    c = lax.axis_index("c")
    s = lax.axis_index("s")
    wid = c * NSUB + s
    rbase = s * RPT

    pltpu.sync_copy(zrows_hbm, shared.at[pl.ds(rbase, RPT)])
    pltpu.sync_copy(h_hbm.at[pl.ds(rbase, RPT)], hbuf)
    pltpu.sync_copy(ei_hbm.at[1, 2 * s], dstv.at[0])
    pltpu.sync_copy(ei_hbm.at[1, 2 * s + 1], dstv.at[1])
    pltpu.sync_copy(ei_hbm.at[0, wid], sidx)
    pltpu.sync_copy(ei_hbm.at[1, wid], didx)

    zeros16 = jnp.zeros((16,), jnp.float32)
    ones16 = jnp.ones((16,), jnp.float32)

    def zero_body(j, carry):
        hist[pl.ds(j * 16, 16)] = zeros16
        return carry
    lax.fori_loop(0, N_PAD // 16, zero_body, 0)

    def count_body(m, carry):
        w = m // NCHUNK
        ch = m % NCHUNK
        for k in range(CW // 16):
            idx = dstv[w, ch, pl.ds(k * 16, 16)]
            plsc.addupdate_scatter(hist, [idx], ones16)
        return carry
    lax.fori_loop(0, 2 * NCHUNK, count_body, 0)

    pltpu.sync_copy(hist, shared_p.at[s])
    plsc.subcore_barrier()

    pltpu.sync_copy(shared_p.at[:, pl.ds(rbase, RPT)], red)

    def dis_body(jj, carry):
        col = jj * 16
        v = red[0, pl.ds(col, 16)]
        for t in range(1, NSUB):
            v = v + red[t, pl.ds(col, 16)]
        disv[pl.ds(col, 16)] = _rsqrt16(v + 1.0)
        return carry
    lax.fori_loop(0, RPT // 16, dis_body, 0)

    def scale_body(jj, carry):
        col = jj * 16
        dv = disv[pl.ds(col, 16)]
        for i in range(16):
            hbuf[col + i] = hbuf[col + i] * dv[i]
            disb[col + i] = ones16 * dv[i]
        return carry
    lax.fori_loop(0, RPT // 16, scale_body, 0)

    @pl.when(c == 0)
    def _():
        pltpu.sync_copy(disb, dis_hbm.at[pl.ds(rbase, RPT)])

    pltpu.sync_copy(hbuf, shared_g.at[pl.ds(rbase, RPT)])
    plsc.subcore_barrier()

    _edge_pipeline(shared_g, shared, sidx, didx, rows, sem_a, sem_b, sem_c)

    plsc.subcore_barrier()
    pltpu.sync_copy(shared.at[pl.ds(rbase, RPT)],
                    out_hbm.at[c, pl.ds(rbase, RPT)])


@functools.cache
def _get_edge_kernel():
    mesh = plsc.VectorSubcoreMesh(
        core_axis_name="c", subcore_axis_name="s",
        num_cores=NCORE, num_subcores=NSUB)
    return functools.partial(
        pl.kernel,
        out_type=jax.ShapeDtypeStruct((NCORE, N_PAD, D_HID), jnp.float32),
        mesh=mesh,
        scratch_types=[
            pltpu.VMEM((NCHUNK, CW), jnp.int32),
            pltpu.VMEM((NCHUNK, CW), jnp.int32),
            pltpu.VMEM((2, NB, CW, D_HID), jnp.float32),
            pltpu.VMEM_SHARED((N_PAD, D_HID), jnp.float32),
            pltpu.VMEM_SHARED((N_PAD, D_HID), jnp.float32),
            pltpu.SemaphoreType.DMA,
            pltpu.SemaphoreType.DMA,
            pltpu.SemaphoreType.DMA,
        ],
        compiler_params=pltpu.CompilerParams(
            needs_layout_passes=False, use_tc_tiling_on_sc=False),
    )(_edge_body)


def _edge_body(g_hbm, ei_hbm, zrows_hbm, out_hbm,
               sidx, didx, rows, shared, shared_g, sem_a, sem_b, sem_c):
    c = lax.axis_index("c")
    s = lax.axis_index("s")
    wid = c * NSUB + s
    rbase = s * RPT

    pltpu.sync_copy(zrows_hbm, shared.at[pl.ds(rbase, RPT)])
    pltpu.sync_copy(g_hbm.at[pl.ds(rbase, RPT)], shared_g.at[pl.ds(rbase, RPT)])
    pltpu.sync_copy(ei_hbm.at[0, wid], sidx)
    pltpu.sync_copy(ei_hbm.at[1, wid], didx)
    plsc.subcore_barrier()

    _edge_pipeline(shared_g, shared, sidx, didx, rows, sem_a, sem_b, sem_c)

    plsc.subcore_barrier()
    pltpu.sync_copy(shared.at[pl.ds(rbase, RPT)],
                    out_hbm.at[c, pl.ds(rbase, RPT)])


def _tc_m_body(x_ref, m1_ref, h_ref):
    h_ref[...] = jnp.dot(x_ref[...], m1_ref[...],
                         preferred_element_type=jnp.float32)


_tc_m = pl.pallas_call(
    _tc_m_body,
    out_shape=jax.ShapeDtypeStruct((NR, 128), jnp.float32),
)


def _tc_b_body(accp_ref, h1_ref, dis_ref, b1_ref, w2k_ref, g2_ref):
    dis = dis_ref[...]
    acc = accp_ref[0] + accp_ref[1] + h1_ref[...] * dis
    t = jnp.maximum(acc * dis + b1_ref[...], 0.0)
    g2_ref[...] = jnp.dot(t, w2k_ref[...],
                          preferred_element_type=jnp.float32) * dis


_tc_b = pl.pallas_call(
    _tc_b_body,
    out_shape=jax.ShapeDtypeStruct((NR, 128), jnp.float32),
)


def _tc_c_body(accp_ref, g2_ref, dis_ref, b2_ref, k0_ref, ks_ref, out_ref):
    acc = accp_ref[0] + accp_ref[1] + g2_ref[...]
    z = acc * dis_ref[...] + b2_ref[...]
    col = lax.broadcasted_iota(jnp.int32, z.shape, 1)
    zm = jnp.where(col % D_HID < N_CLASSES, z, -jnp.inf)
    y = zm
    for sft in (1, 2, 4, 8):
        y = jnp.maximum(y, jnp.roll(y, -sft, axis=1))
    mask0 = (col % D_HID == 0).astype(jnp.float32)
    m = jnp.dot(y * mask0, k0_ref[...], preferred_element_type=jnp.float32)
    e = jnp.exp(zm - m)
    se = jnp.dot(e, ks_ref[...], preferred_element_type=jnp.float32)
    out_ref[...] = z - m - jnp.log(se)


_tc_c = pl.pallas_call(
    _tc_c_body,
    out_shape=jax.ShapeDtypeStruct((NR, 128), jnp.float32),
)


def kernel(x, edge_index, W1, b1, W2, b2):
    eye8 = jnp.eye(8, dtype=jnp.float32)
    ei = jnp.concatenate(
        [edge_index.astype(jnp.int32),
         jnp.full((2, E_PAD - N_EDGES), N_NODES, jnp.int32)],
        axis=1).reshape(2, NW, NCHUNK, CW)
    x1024 = jnp.pad(x, ((0, N_PAD - N_NODES), (0, 0))).reshape(NR, 8 * D_FEAT)
    m1 = jnp.kron(eye8, W1)
    w2k = jnp.kron(eye8, jnp.pad(W2, ((0, 0), (0, D_HID - N_CLASSES))))
    b1t = jnp.tile(b1.reshape(1, D_HID), (1, 8))
    b2t = jnp.tile(jnp.pad(b2, (0, D_HID - N_CLASSES)).reshape(1, D_HID),
                   (1, 8))
    zrows = jnp.zeros((RPT, D_HID), jnp.float32)

    e0 = jnp.zeros((D_HID, D_HID), jnp.float32).at[0].set(1.0)
    k0 = jnp.kron(eye8, e0)
    ks = jnp.kron(eye8, jnp.ones((D_HID, D_HID), jnp.float32))

    h1_128 = _tc_m(x1024, m1)
    accp1, dis16 = _get_l1_kernel()(h1_128.reshape(N_PAD, D_HID), ei, zrows)
    dis128 = dis16.reshape(NR, 128)
    g2_128 = _tc_b(accp1.reshape(NCORE, NR, 128), h1_128, dis128, b1t, w2k)
    accp2 = _get_edge_kernel()(g2_128.reshape(N_PAD, D_HID), ei, zrows)
    z128 = _tc_c(accp2.reshape(NCORE, NR, 128), g2_128, dis128, b2t, k0, ks)
    z = z128.reshape(N_PAD, D_HID)
    return z[:N_NODES, :N_CLASSES]


# speedup vs baseline: 1.9008x; 1.0130x over previous
import functools

import jax
import jax.numpy as jnp
from jax import lax
from jax.experimental import pallas as pl
from jax.experimental.pallas import tpu as pltpu
from jax.experimental.pallas import tpu_sc as plsc

N_NODES = 10000
N_PAD = 10240
N_EDGES = 320000
E_PAD = 327680
D_FEAT = 128
D_HID = 16
N_CLASSES = 7

NCORE = 2
NSUB = 16
NW = NCORE * NSUB
EPW = E_PAD // NW
NCHUNK = 80
CW = 128
NB = 4
NGROUP = NCHUNK // NB
RPT = N_PAD // NSUB
NR = N_PAD * D_HID // 128


def _edge_pipeline(shared_g, shared, sidx, didx, rows, sem_a, sem_b, sem_c):
    def _fire(g, half, sem):
        for b in range(NB):
            pltpu.async_copy(shared_g.at[sidx.at[g * NB + b]],
                             rows.at[half, b], sem)

    def _drain_scatter(g, half, sem):
        for b in range(NB):
            pltpu.make_async_copy(shared_g.at[pl.ds(0, CW)],
                                  rows.at[half, b], sem).wait()
        for b in range(NB):
            pltpu.async_copy(rows.at[half, b],
                             shared.at[didx.at[g * NB + b]], sem_c, add=True)
        for b in range(NB):
            pltpu.make_async_copy(shared_g.at[pl.ds(0, CW)],
                                  rows.at[half, b], sem_c).wait()

    _fire(0, 0, sem_a)

    def body(kk, carry):
        g0 = 2 * kk
        _fire(g0 + 1, 1, sem_b)
        _drain_scatter(g0, 0, sem_a)

        @pl.when(g0 + 2 < NGROUP)
        def _():
            _fire(g0 + 2, 0, sem_a)
        _drain_scatter(g0 + 1, 1, sem_b)
        return carry
    lax.fori_loop(0, NGROUP // 2, body, 0)


def _rsqrt16(v):
    xi = lax.bitcast_convert_type(v, jnp.int32)
    yi = jnp.int32(0x5F3759DF) - (xi >> 1)
    y = lax.bitcast_convert_type(yi, jnp.float32)
    for _ in range(3):
        y = y * (1.5 - 0.5 * v * y * y)
    return y


@functools.cache
def _get_l1_kernel():
    mesh = plsc.VectorSubcoreMesh(
        core_axis_name="c", subcore_axis_name="s",
        num_cores=NCORE, num_subcores=NSUB)
    return functools.partial(
        pl.kernel,
        out_type=(jax.ShapeDtypeStruct((NCORE, N_PAD, D_HID), jnp.float32),
                  jax.ShapeDtypeStruct((N_PAD, D_HID), jnp.float32)),
        mesh=mesh,
        scratch_types=[
            pltpu.VMEM((NCHUNK, CW), jnp.int32),
            pltpu.VMEM((NCHUNK, CW), jnp.int32),
            pltpu.VMEM((2, NB, CW, D_HID), jnp.float32),
            pltpu.VMEM((2, NCHUNK, CW), jnp.int32),
            pltpu.VMEM((RPT, D_HID), jnp.float32),
            pltpu.VMEM((NSUB, RPT // 16, D_HID), jnp.float32),
            pltpu.VMEM((RPT,), jnp.float32),
            pltpu.VMEM((RPT, D_HID), jnp.float32),
            pltpu.VMEM((RPT, D_HID), jnp.float32),
            pltpu.VMEM_SHARED((N_PAD, D_HID), jnp.float32),
            pltpu.VMEM_SHARED((N_PAD, D_HID), jnp.float32),
            pltpu.VMEM_SHARED((NSUB, RPT, D_HID), jnp.float32),
            pltpu.SemaphoreType.DMA,
            pltpu.SemaphoreType.DMA,
            pltpu.SemaphoreType.DMA,
        ],
        compiler_params=pltpu.CompilerParams(
            needs_layout_passes=False, use_tc_tiling_on_sc=False),
    )(_l1_body)


def _l1_body(h_hbm, ei_hbm, zrows_hbm,
             out_hbm, dis_hbm,
             sidx, didx, rows, dstv, hist, red, disv, hbuf, disb,
             shared, shared_g, shared_p, sem_a, sem_b, sem_c):
    c = lax.axis_index("c")
    s = lax.axis_index("s")
    wid = c * NSUB + s
    rbase = s * RPT

    pltpu.sync_copy(zrows_hbm, shared.at[pl.ds(rbase, RPT)])
    pltpu.sync_copy(h_hbm.at[pl.ds(rbase, RPT)], hbuf)
    pltpu.sync_copy(ei_hbm.at[1, 2 * s], dstv.at[0])
    pltpu.sync_copy(ei_hbm.at[1, 2 * s + 1], dstv.at[1])
    pltpu.sync_copy(ei_hbm.at[0, wid], sidx)
    pltpu.sync_copy(ei_hbm.at[1, wid], didx)
    pltpu.sync_copy(zrows_hbm, hist)

    ones16 = jnp.ones((16,), jnp.float32)

    for w in range(2):
        def count_body(ch, carry):
            for k in range(CW // 16):
                idx = dstv[w, ch, pl.ds(k * 16, 16)]
                plsc.addupdate_scatter(hist, [idx >> 4, idx & 15], ones16)
            return carry
        lax.fori_loop(0, NCHUNK, count_body, 0)

    pltpu.sync_copy(hist, shared_p.at[s])
    plsc.subcore_barrier()

    pltpu.sync_copy(shared_p.at[:, pl.ds(s * (RPT // 16), RPT // 16)], red)

    def dis_body(jj, carry):
        v = red[0, jj]
        for t in range(1, NSUB):
            v = v + red[t, jj]
        disv[pl.ds(jj * 16, 16)] = _rsqrt16(v + 1.0)
        return carry
    lax.fori_loop(0, RPT // 16, dis_body, 0)

    def scale_body(jj, carry):
        col = jj * 16
        dv = disv[pl.ds(col, 16)]
        for i in range(16):
            hbuf[col + i] = hbuf[col + i] * dv[i]
            disb[col + i] = ones16 * dv[i]
        return carry
    lax.fori_loop(0, RPT // 16, scale_body, 0)

    @pl.when(c == 0)
    def _():
        pltpu.sync_copy(disb, dis_hbm.at[pl.ds(rbase, RPT)])

    pltpu.sync_copy(hbuf, shared_g.at[pl.ds(rbase, RPT)])
    plsc.subcore_barrier()

    _edge_pipeline(shared_g, shared, sidx, didx, rows, sem_a, sem_b, sem_c)

    plsc.subcore_barrier()
    pltpu.sync_copy(shared.at[pl.ds(rbase, RPT)],
                    out_hbm.at[c, pl.ds(rbase, RPT)])


@functools.cache
def _get_edge_kernel():
    mesh = plsc.VectorSubcoreMesh(
        core_axis_name="c", subcore_axis_name="s",
        num_cores=NCORE, num_subcores=NSUB)
    return functools.partial(
        pl.kernel,
        out_type=jax.ShapeDtypeStruct((NCORE, N_PAD, D_HID), jnp.float32),
        mesh=mesh,
        scratch_types=[
            pltpu.VMEM((NCHUNK, CW), jnp.int32),
            pltpu.VMEM((NCHUNK, CW), jnp.int32),
            pltpu.VMEM((2, NB, CW, D_HID), jnp.float32),
            pltpu.VMEM_SHARED((N_PAD, D_HID), jnp.float32),
            pltpu.VMEM_SHARED((N_PAD, D_HID), jnp.float32),
            pltpu.SemaphoreType.DMA,
            pltpu.SemaphoreType.DMA,
            pltpu.SemaphoreType.DMA,
        ],
        compiler_params=pltpu.CompilerParams(
            needs_layout_passes=False, use_tc_tiling_on_sc=False),
    )(_edge_body)


def _edge_body(g_hbm, ei_hbm, zrows_hbm, out_hbm,
               sidx, didx, rows, shared, shared_g, sem_a, sem_b, sem_c):
    c = lax.axis_index("c")
    s = lax.axis_index("s")
    wid = c * NSUB + s
    rbase = s * RPT

    pltpu.sync_copy(zrows_hbm, shared.at[pl.ds(rbase, RPT)])
    pltpu.sync_copy(g_hbm.at[pl.ds(rbase, RPT)], shared_g.at[pl.ds(rbase, RPT)])
    pltpu.sync_copy(ei_hbm.at[0, wid], sidx)
    pltpu.sync_copy(ei_hbm.at[1, wid], didx)
    plsc.subcore_barrier()

    _edge_pipeline(shared_g, shared, sidx, didx, rows, sem_a, sem_b, sem_c)

    plsc.subcore_barrier()
    pltpu.sync_copy(shared.at[pl.ds(rbase, RPT)],
                    out_hbm.at[c, pl.ds(rbase, RPT)])


def _tc_m_body(x_ref, m1_ref, h_ref):
    h_ref[...] = jnp.dot(x_ref[...], m1_ref[...],
                         preferred_element_type=jnp.float32)


_tc_m = pl.pallas_call(
    _tc_m_body,
    out_shape=jax.ShapeDtypeStruct((NR, 128), jnp.float32),
)


def _tc_b_body(accp_ref, h1_ref, dis_ref, b1_ref, w2k_ref, g2_ref):
    dis = dis_ref[...]
    acc = accp_ref[0] + accp_ref[1] + h1_ref[...] * dis
    t = jnp.maximum(acc * dis + b1_ref[...], 0.0)
    g2_ref[...] = jnp.dot(t, w2k_ref[...],
                          preferred_element_type=jnp.float32) * dis


_tc_b = pl.pallas_call(
    _tc_b_body,
    out_shape=jax.ShapeDtypeStruct((NR, 128), jnp.float32),
)


def _tc_c_body(accp_ref, g2_ref, dis_ref, b2_ref, k0_ref, ks_ref, out_ref):
    acc = accp_ref[0] + accp_ref[1] + g2_ref[...]
    z = acc * dis_ref[...] + b2_ref[...]
    col = lax.broadcasted_iota(jnp.int32, z.shape, 1)
    zm = jnp.where(col % D_HID < N_CLASSES, z, -jnp.inf)
    y = zm
    for sft in (1, 2, 4, 8):
        y = jnp.maximum(y, jnp.roll(y, -sft, axis=1))
    mask0 = (col % D_HID == 0).astype(jnp.float32)
    m = jnp.dot(y * mask0, k0_ref[...], preferred_element_type=jnp.float32)
    e = jnp.exp(zm - m)
    se = jnp.dot(e, ks_ref[...], preferred_element_type=jnp.float32)
    out_ref[...] = z - m - jnp.log(se)


_tc_c = pl.pallas_call(
    _tc_c_body,
    out_shape=jax.ShapeDtypeStruct((NR, 128), jnp.float32),
)


def kernel(x, edge_index, W1, b1, W2, b2):
    eye8 = jnp.eye(8, dtype=jnp.float32)
    ei = jnp.concatenate(
        [edge_index.astype(jnp.int32),
         jnp.full((2, E_PAD - N_EDGES), N_NODES, jnp.int32)],
        axis=1).reshape(2, NW, NCHUNK, CW)
    x1024 = jnp.pad(x, ((0, N_PAD - N_NODES), (0, 0))).reshape(NR, 8 * D_FEAT)
    m1 = jnp.kron(eye8, W1)
    w2k = jnp.kron(eye8, jnp.pad(W2, ((0, 0), (0, D_HID - N_CLASSES))))
    b1t = jnp.tile(b1.reshape(1, D_HID), (1, 8))
    b2t = jnp.tile(jnp.pad(b2, (0, D_HID - N_CLASSES)).reshape(1, D_HID),
                   (1, 8))
    zrows = jnp.zeros((RPT, D_HID), jnp.float32)

    e0 = jnp.zeros((D_HID, D_HID), jnp.float32).at[0].set(1.0)
    k0 = jnp.kron(eye8, e0)
    ks = jnp.kron(eye8, jnp.ones((D_HID, D_HID), jnp.float32))

    h1_128 = _tc_m(x1024, m1)
    accp1, dis16 = _get_l1_kernel()(h1_128.reshape(N_PAD, D_HID), ei, zrows)
    dis128 = dis16.reshape(NR, 128)
    g2_128 = _tc_b(accp1.reshape(NCORE, NR, 128), h1_128, dis128, b1t, w2k)
    accp2 = _get_edge_kernel()(g2_128.reshape(N_PAD, D_HID), ei, zrows)
    z128 = _tc_c(accp2.reshape(NCORE, NR, 128), g2_128, dis128, b2t, k0, ks)
    z = z128.reshape(N_PAD, D_HID)
    return z[:N_NODES, :N_CLASSES]
